# Initial kernel scaffold; baseline (speedup 1.0000x reference)
import jax
import jax.numpy as jnp
from jax.experimental import pallas as pl


def kernel(pairs, k, u, beta):
    raise NotImplementedError("write your pallas kernel here")


# speedup vs baseline: 5.2688x; 5.2688x over previous
import functools

import jax
import jax.numpy as jnp
from jax import lax
from jax.experimental import pallas as pl
from jax.experimental.pallas import tpu as pltpu
from jax.experimental.pallas import tpu_sc as plsc

N_PAIRS = 16384
M_PAD = 1024
L = 16
NC, NS = 2, 16
NW = NC * NS
PAIRS_PER_W = N_PAIRS // NW
WORDS_PER_W = 2 * PAIRS_PER_W
CHUNKS = PAIRS_PER_W // L

_LN2 = 0.6931471805599453
_SQRT2 = 1.4142135


def _ln(y):
    yi = lax.bitcast_convert_type(y, jnp.int32)
    e = (yi >> 23) - 127
    m = lax.bitcast_convert_type((yi & 0x7FFFFF) | 0x3F800000, jnp.float32)
    big = m > _SQRT2
    m = jnp.where(big, m * 0.5, m)
    ef = (e + big.astype(jnp.int32)).astype(jnp.float32)
    t = (m - 1.0) / (m + 1.0)
    t2 = t * t
    p = 2.0 * t * (1.0 + t2 * (1.0 / 3.0 + t2 * (0.2 + t2 * (1.0 / 7.0))))
    return ef * _LN2 + p


def _body(pairs_hbm, u_hbm, b_hbm, out_hbm, pairs_v, u_v, b_v, acc_v):
    wid = lax.axis_index("s") * NC + lax.axis_index("c")
    base = wid * WORDS_PER_W
    pltpu.sync_copy(pairs_hbm.at[pl.ds(base, WORDS_PER_W)], pairs_v)
    pltpu.sync_copy(u_hbm, u_v)
    pltpu.sync_copy(b_hbm, b_v)

    b = b_v[...]
    even = jnp.arange(L, dtype=jnp.int32) * 2
    acc = jnp.zeros((L,), jnp.float32)
    for j in range(CHUNKS):
        iw = even + (j * 2 * L)
        w_idx = plsc.load_gather(pairs_v, [iw])
        l_idx = plsc.load_gather(pairs_v, [iw + 1])
        uw = plsc.load_gather(u_v, [w_idx])
        ul = plsc.load_gather(u_v, [l_idx])
        y = 1.0 + jnp.exp(b * (ul - uw))
        acc = acc + _ln(y)
    acc_v[...] = acc
    pltpu.sync_copy(acc_v, out_hbm.at[wid])


_sc_call = pl.kernel(
    _body,
    out_type=jax.ShapeDtypeStruct((NW, L), jnp.float32),
    mesh=plsc.VectorSubcoreMesh(core_axis_name="c", subcore_axis_name="s"),
    compiler_params=pltpu.CompilerParams(needs_layout_passes=False),
    scratch_types=[
        pltpu.VMEM((WORDS_PER_W,), jnp.int32),
        pltpu.VMEM((M_PAD,), jnp.float32),
        pltpu.VMEM((L,), jnp.float32),
        pltpu.VMEM((L,), jnp.float32),
    ],
)


def kernel(pairs, k, u, beta):
    b = jnp.where(k == 0, jnp.float32(1.0), beta[k]).astype(jnp.float32)
    b16 = jnp.full((L,), b, jnp.float32)
    pairs_flat = pairs.reshape(-1)
    u_pad = jnp.zeros((M_PAD,), jnp.float32).at[: u.shape[0]].set(u)
    partials = _sc_call(pairs_flat, u_pad, b16)
    return jnp.sum(partials)


# speedup vs baseline: 5.3959x; 1.0241x over previous
import functools

import jax
import jax.numpy as jnp
from jax import lax
from jax.experimental import pallas as pl
from jax.experimental.pallas import tpu as pltpu
from jax.experimental.pallas import tpu_sc as plsc

N_PAIRS = 16384
M_PAD = 1024
L = 16
NC, NS = 2, 16
NW = NC * NS
PAIRS_PER_W = N_PAIRS // NW
WORDS_PER_W = 2 * PAIRS_PER_W
CHUNKS = PAIRS_PER_W // L
AUX = M_PAD + L

_LN2 = 0.6931471805599453
_SQRT2 = 1.4142135


def _ln(y):
    yi = lax.bitcast_convert_type(y, jnp.int32)
    e = (yi >> 23) - 127
    m = lax.bitcast_convert_type((yi & 0x7FFFFF) | 0x3F800000, jnp.float32)
    big = m > _SQRT2
    m = jnp.where(big, m * 0.5, m)
    ef = (e + big.astype(jnp.int32)).astype(jnp.float32)
    t = (m - 1.0) / (m + 1.0)
    t2 = t * t
    p = 2.0 * t * (1.0 + t2 * (1.0 / 3.0 + t2 * (0.2 + t2 * (1.0 / 7.0))))
    return ef * _LN2 + p


def _body(pairs_hbm, aux_hbm, out_hbm, pairs_v, aux_v, acc_v):
    wid = lax.axis_index("s") * NC + lax.axis_index("c")
    base = wid * WORDS_PER_W
    pltpu.sync_copy(pairs_hbm.at[pl.ds(base, WORDS_PER_W)], pairs_v)
    pltpu.sync_copy(aux_hbm, aux_v)

    b = aux_v[pl.ds(M_PAD, L)]
    even = jnp.arange(L, dtype=jnp.int32) * 2
    acc = jnp.zeros((L,), jnp.float32)
    for j in range(CHUNKS):
        iw = even + (j * 2 * L)
        w_idx = plsc.load_gather(pairs_v, [iw])
        l_idx = plsc.load_gather(pairs_v, [iw + 1])
        uw = plsc.load_gather(aux_v, [w_idx])
        ul = plsc.load_gather(aux_v, [l_idx])
        y = 1.0 + jnp.exp(b * (ul - uw))
        acc = acc + _ln(y)
    acc_v[...] = acc
    pltpu.sync_copy(acc_v, out_hbm.at[wid])


_sc_call = pl.kernel(
    _body,
    out_type=jax.ShapeDtypeStruct((NW, L), jnp.float32),
    mesh=plsc.VectorSubcoreMesh(core_axis_name="c", subcore_axis_name="s"),
    compiler_params=pltpu.CompilerParams(needs_layout_passes=False),
    scratch_types=[
        pltpu.VMEM((WORDS_PER_W,), jnp.int32),
        pltpu.VMEM((AUX,), jnp.float32),
        pltpu.VMEM((L,), jnp.float32),
    ],
)


def kernel(pairs, k, u, beta):
    b = jnp.where(k == 0, jnp.float32(1.0), beta[k]).astype(jnp.float32)
    aux = jnp.concatenate([
        u,
        jnp.zeros((M_PAD - u.shape[0],), jnp.float32),
        jnp.full((L,), b, jnp.float32),
    ])
    partials = _sc_call(pairs.reshape(-1), aux)
    return jnp.sum(partials)


# speedup vs baseline: 6.1180x; 1.1338x over previous
import functools

import jax
import jax.numpy as jnp
from jax import lax
from jax.experimental import pallas as pl
from jax.experimental.pallas import tpu as pltpu
from jax.experimental.pallas import tpu_sc as plsc

N_PAIRS = 16384
M_PAD = 1024
L = 16
NC, NS = 2, 16
NW = NC * NS
PAIRS_PER_W = N_PAIRS // NW
WORDS_PER_W = 2 * PAIRS_PER_W
CHUNKS = PAIRS_PER_W // L
AUX = M_PAD + L

_LN2 = 0.6931471805599453
_SQRT2 = 1.4142135


def _ln(y):
    yi = lax.bitcast_convert_type(y, jnp.int32)
    e = (yi >> 23) - 127
    m = lax.bitcast_convert_type((yi & 0x7FFFFF) | 0x3F800000, jnp.float32)
    big = m > _SQRT2
    m = jnp.where(big, m * 0.5, m)
    ef = (e + big.astype(jnp.int32)).astype(jnp.float32)
    t = (m - 1.0) / (m + 1.0)
    t2 = t * t
    p = 2.0 * t * (1.0 + t2 * (1.0 / 3.0 + t2 * (0.2 + t2 * (1.0 / 7.0))))
    return ef * _LN2 + p


def _body(pairs_hbm, aux_hbm, out_hbm, pairs_v, aux_v, acc_v):
    wid = lax.axis_index("s") * NC + lax.axis_index("c")
    row0 = wid * PAIRS_PER_W
    pltpu.sync_copy(pairs_hbm.at[pl.ds(row0, PAIRS_PER_W)], pairs_v)
    pltpu.sync_copy(aux_hbm, aux_v)

    b = aux_v[pl.ds(M_PAD, L)]
    lane = jnp.arange(L, dtype=jnp.int32)
    zeros = jnp.zeros((L,), jnp.int32)
    ones = zeros + 1
    acc = jnp.zeros((L,), jnp.float32)
    for j in range(CHUNKS):
        rows = lane + (j * L)
        w_idx = plsc.load_gather(pairs_v, [rows, zeros])
        l_idx = plsc.load_gather(pairs_v, [rows, ones])
        uw = plsc.load_gather(aux_v, [w_idx])
        ul = plsc.load_gather(aux_v, [l_idx])
        y = 1.0 + jnp.exp(b * (ul - uw))
        acc = acc + _ln(y)
    acc_v[...] = acc
    pltpu.sync_copy(acc_v, out_hbm.at[wid])


_sc_call = pl.kernel(
    _body,
    out_type=jax.ShapeDtypeStruct((NW, L), jnp.float32),
    mesh=plsc.VectorSubcoreMesh(core_axis_name="c", subcore_axis_name="s"),
    compiler_params=pltpu.CompilerParams(needs_layout_passes=False),
    scratch_types=[
        pltpu.VMEM((PAIRS_PER_W, 2), jnp.int32),
        pltpu.VMEM((AUX,), jnp.float32),
        pltpu.VMEM((L,), jnp.float32),
    ],
)


def kernel(pairs, k, u, beta):
    b = jnp.where(k == 0, jnp.float32(1.0), beta[k]).astype(jnp.float32)
    aux = jnp.concatenate([
        u,
        jnp.zeros((M_PAD - u.shape[0],), jnp.float32),
        jnp.full((L,), b, jnp.float32),
    ])
    partials = _sc_call(pairs, aux)
    return jnp.sum(partials)


# speedup vs baseline: 6.2591x; 1.0231x over previous
import functools

import jax
import jax.numpy as jnp
from jax import lax
from jax.experimental import pallas as pl
from jax.experimental.pallas import tpu as pltpu
from jax.experimental.pallas import tpu_sc as plsc

N_PAIRS = 16384
M_PAD = 1024
L = 16
NC, NS = 2, 16
NW = NC * NS
PAIRS_PER_W = N_PAIRS // NW
WORDS_PER_W = 2 * PAIRS_PER_W
CHUNKS = PAIRS_PER_W // L
AUX = M_PAD + L

_LN2 = 0.6931471805599453
_SQRT2 = 1.4142135


def _ln(y):
    yi = lax.bitcast_convert_type(y, jnp.int32)
    e = (yi >> 23) - 127
    m = lax.bitcast_convert_type((yi & 0x7FFFFF) | 0x3F800000, jnp.float32)
    big = m > _SQRT2
    m = jnp.where(big, m * 0.5, m)
    ef = (e + big.astype(jnp.int32)).astype(jnp.float32)
    t = (m - 1.0) / (m + 1.0)
    t2 = t * t
    p = 2.0 * t * (1.0 + t2 * (1.0 / 3.0 + t2 * (0.2 + t2 * (1.0 / 7.0))))
    return ef * _LN2 + p


def _body(pairs_hbm, aux_hbm, out_hbm, pairs_v, aux_v, acc_v):
    wid = lax.axis_index("s") * NC + lax.axis_index("c")
    row0 = wid * PAIRS_PER_W
    pltpu.sync_copy(pairs_hbm.at[pl.ds(row0, PAIRS_PER_W)], pairs_v)
    pltpu.sync_copy(aux_hbm, aux_v)

    b = aux_v[pl.ds(M_PAD, L)]
    lane = jnp.arange(L, dtype=jnp.int32)
    zeros = jnp.zeros((L,), jnp.int32)
    ones = zeros + 1

    def chunk(j, acc):
        rows = lane + (j * L)
        w_idx = plsc.load_gather(pairs_v, [rows, zeros])
        l_idx = plsc.load_gather(pairs_v, [rows, ones])
        uw = plsc.load_gather(aux_v, [w_idx])
        ul = plsc.load_gather(aux_v, [l_idx])
        y = 1.0 + jnp.exp(b * (ul - uw))
        return acc + _ln(y)

    acc_v[...] = lax.fori_loop(0, CHUNKS, chunk, jnp.zeros((L,), jnp.float32))
    pltpu.sync_copy(acc_v, out_hbm.at[wid])


_sc_call = pl.kernel(
    _body,
    out_type=jax.ShapeDtypeStruct((NW, L), jnp.float32),
    mesh=plsc.VectorSubcoreMesh(core_axis_name="c", subcore_axis_name="s"),
    compiler_params=pltpu.CompilerParams(needs_layout_passes=False),
    scratch_types=[
        pltpu.VMEM((PAIRS_PER_W, 2), jnp.int32),
        pltpu.VMEM((AUX,), jnp.float32),
        pltpu.VMEM((L,), jnp.float32),
    ],
)


def kernel(pairs, k, u, beta):
    b = jnp.where(k == 0, jnp.float32(1.0), beta[k]).astype(jnp.float32)
    aux = jnp.concatenate([
        u,
        jnp.zeros((M_PAD - u.shape[0],), jnp.float32),
        jnp.full((L,), b, jnp.float32),
    ])
    partials = _sc_call(pairs, aux)
    return jnp.sum(partials)
